# Initial kernel scaffold; baseline (speedup 1.0000x reference)
import jax
import jax.numpy as jnp
from jax.experimental import pallas as pl


def kernel(x, target, y, idx, trainLabel, memory):
    raise NotImplementedError("write your pallas kernel here")


# speedup vs baseline: 13.4145x; 13.4145x over previous
import functools

import jax
import jax.numpy as jnp
from jax import lax
from jax.experimental import pallas as pl
from jax.experimental.pallas import tpu as pltpu

BS = 1024
IN = 128
OUT = 100000
K = 4096
T = 0.07
MOMENTUM = 0.5

TN = 2048
NT = 49
NCHT = TN // 128
NCH = NT * NCHT
KP = 33 * 128
NEG = -1e30


def _sim_body(x_ref, m_ref, out_ref, cmax_ref):
    t = pl.program_id(0)
    tile = jax.lax.dot_general(
        x_ref[...], m_ref[...], (((1,), (1,)), ((), ())),
        preferred_element_type=jnp.float32,
        precision=jax.lax.Precision.DEFAULT)
    col = jax.lax.broadcasted_iota(jnp.int32, (BS, TN), 1) + t * TN
    tile = jnp.where(col < OUT, tile, NEG)
    out_ref[0] = tile
    for c in range(NCHT):
        cmax_ref[0, c, :] = jnp.max(tile[:, c * 128:(c + 1) * 128], axis=1)


def _similarity(x, memory):
    return pl.pallas_call(
        _sim_body,
        grid=(NT,),
        in_specs=[
            pl.BlockSpec((BS, IN), lambda t: (0, 0)),
            pl.BlockSpec((TN, IN), lambda t: (t, 0)),
        ],
        out_specs=[
            pl.BlockSpec((1, BS, TN), lambda t: (t, 0, 0)),
            pl.BlockSpec((1, NCHT, BS), lambda t: (t, 0, 0)),
        ],
        out_shape=[
            jax.ShapeDtypeStruct((NT, BS, TN), jnp.float32),
            jax.ShapeDtypeStruct((NT, NCHT, BS), jnp.float32),
        ],
    )(x, memory)


def _topchunk_body(cm_ref, cid_ref):
    v = cm_ref[...].reshape(NCH, BS)
    ii = jax.lax.broadcasted_iota(jnp.int32, (NCH, BS), 0)
    for k in range(32):
        m = jnp.max(v, axis=0)
        sel = jnp.min(jnp.where(v == m[None, :], ii, NCH), axis=0)
        cid_ref[k, :] = sel
        v = jnp.where(ii == sel[None, :], -jnp.inf, v)


def _topchunks(cmax):
    return pl.pallas_call(
        _topchunk_body,
        out_shape=jax.ShapeDtypeStruct((32, BS), jnp.int32),
    )(cmax)


def _topk_body(cand_ref, cols_ref, yd_ref, yi_ref):
    v = cand_ref[...]
    cols = cols_ref[...]
    for k in range(32):
        m = jnp.max(v, axis=1)
        sel = jnp.min(jnp.where(v == m[:, None], cols, jnp.int32(2**30)), axis=1)
        yd_ref[:, k] = m
        yi_ref[:, k] = sel
        v = jnp.where(cols == sel[:, None], -jnp.inf, v)


def _topk(cand, cols):
    return pl.pallas_call(
        _topk_body,
        out_shape=[
            jax.ShapeDtypeStruct((BS, 32), jnp.float32),
            jax.ShapeDtypeStruct((BS, 32), jnp.int32),
        ],
    )(cand, cols)


def _exp_body(nce_ref, e_ref, rs_ref):
    col = jax.lax.broadcasted_iota(jnp.int32, (BS, KP), 1)
    v = jnp.where(col <= K, nce_ref[...], -jnp.inf)
    e = jnp.exp(v * jnp.float32(1.0 / T))
    e_ref[...] = e
    rs_ref[...] = jnp.sum(e, axis=1, keepdims=True)


def _exp_norm(nce_pad):
    return pl.pallas_call(
        _exp_body,
        out_shape=[
            jax.ShapeDtypeStruct((BS, KP), jnp.float32),
            jax.ShapeDtypeStruct((BS, 1), jnp.float32),
        ],
    )(nce_pad)


def _norm_body(my_ref, xw_ref, o_ref):
    w = my_ref[...] * jnp.float32(MOMENTUM) + xw_ref[...] * jnp.float32(1.0 - MOMENTUM)
    n = jnp.maximum(jnp.sqrt(jnp.sum(w * w, axis=1, keepdims=True)), 1e-12)
    o_ref[...] = w / n


def _mix_norm(mem_y, xw):
    return pl.pallas_call(
        _norm_body,
        out_shape=jax.ShapeDtypeStruct((BS, IN), jnp.float32),
    )(mem_y, xw)


def kernel(x, target, y, idx, trainLabel, memory):
    out_full, cmax = _similarity(x, memory)

    chunk_ids = _topchunks(cmax)
    cid_t = chunk_ids.T

    tt = cid_t // NCHT
    ci = cid_t % NCHT
    rows = (tt * BS + jnp.arange(BS, dtype=jnp.int32)[:, None]) * NCHT + ci
    cand = jnp.take(out_full.reshape(NT * BS * NCHT, 128), rows.reshape(-1), axis=0)
    cand = cand.reshape(BS, 32 * 128)
    cols = (cid_t[:, :, None] * 128
            + jnp.arange(128, dtype=jnp.int32)[None, None, :]).reshape(BS, 32 * 128)

    yd, yi = _topk(cand, cols)
    retrieval = jnp.take(trainLabel, yi.reshape(-1), axis=0).reshape(BS, 32)

    t_i = idx >> 11
    c_i = idx & (TN - 1)
    flat = t_i * (BS * TN) + jnp.arange(BS, dtype=jnp.int32)[:, None] * TN + c_i
    nce = jnp.take(out_full.reshape(-1), flat.reshape(-1), axis=0).reshape(BS, K + 1)
    nce_pad = jnp.pad(nce, ((0, 0), (0, KP - (K + 1))))

    e, rowsum = _exp_norm(nce_pad)
    total = jnp.sum(rowsum)
    Z = total / jnp.float32(BS * (K + 1)) * jnp.float32(OUT)
    out = e[:, :K + 1] / Z
    probs = jnp.mean(e[:, 0] / rowsum[:, 0])

    iarange = jnp.arange(BS, dtype=jnp.int32)
    winner = jnp.argmax(jnp.where(y[None, :] == y[:, None], iarange[None, :], -1),
                        axis=1).astype(jnp.int32)
    xw = jnp.take(x, winner, axis=0)
    mem_y = jnp.take(memory, y, axis=0)
    normed = _mix_norm(mem_y, xw)
    new_memory = memory.at[y].set(normed, unique_indices=False)

    return out, probs, yd, retrieval, new_memory


# speedup vs baseline: 29.8774x; 2.2272x over previous
import functools

import jax
import jax.numpy as jnp
from jax import lax
from jax.experimental import pallas as pl
from jax.experimental.pallas import tpu as pltpu
from jax.experimental.pallas import tpu_sc as plsc

BS = 1024
IN = 128
OUT = 100000
K = 4096
T = 0.07
MOMENTUM = 0.5

TN = 2048
NT = 49
NCHT = TN // 128
NCH = NT * NCHT
KP = 33 * 128
NEG = -1e30


def _sim_body(x_ref, m_ref, out_ref, cmax_ref):
    t = pl.program_id(0)
    tile = jax.lax.dot_general(
        x_ref[...], m_ref[...], (((1,), (1,)), ((), ())),
        preferred_element_type=jnp.float32,
        precision=jax.lax.Precision.DEFAULT)
    col = jax.lax.broadcasted_iota(jnp.int32, (BS, TN), 1) + t * TN
    tile = jnp.where(col < OUT, tile, NEG)
    out_ref[...] = tile.reshape(BS * NCHT, 128)
    for c in range(NCHT):
        cmax_ref[0, c, :] = jnp.max(tile[:, c * 128:(c + 1) * 128], axis=1)


def _similarity(x, memory):
    return pl.pallas_call(
        _sim_body,
        grid=(NT,),
        in_specs=[
            pl.BlockSpec((BS, IN), lambda t: (0, 0)),
            pl.BlockSpec((TN, IN), lambda t: (t, 0)),
        ],
        out_specs=[
            pl.BlockSpec((BS * NCHT, 128), lambda t: (t, 0)),
            pl.BlockSpec((1, NCHT, BS), lambda t: (t, 0, 0)),
        ],
        out_shape=[
            jax.ShapeDtypeStruct((NT * BS * NCHT, 128), jnp.float32),
            jax.ShapeDtypeStruct((NT, NCHT, BS), jnp.float32),
        ],
    )(x, memory)


def _topchunk_body(cm_ref, cid_ref):
    v = cm_ref[...].reshape(NCH, BS)
    ii = jax.lax.broadcasted_iota(jnp.int32, (NCH, BS), 0)
    for k in range(32):
        m = jnp.max(v, axis=0)
        sel = jnp.min(jnp.where(v == m[None, :], ii, NCH), axis=0)
        cid_ref[k, :] = sel
        v = jnp.where(ii == sel[None, :], -jnp.inf, v)


def _topchunks(cmax):
    return pl.pallas_call(
        _topchunk_body,
        out_shape=jax.ShapeDtypeStruct((32, BS), jnp.int32),
    )(cmax)


def _topk_body(cand_ref, cols_ref, yd_ref, yi_ref):
    v = cand_ref[...]
    cols = cols_ref[...]
    for k in range(32):
        m = jnp.max(v, axis=1)
        sel = jnp.min(jnp.where(v == m[:, None], cols, jnp.int32(2**30)), axis=1)
        yd_ref[:, k] = m
        yi_ref[:, k] = sel
        v = jnp.where(cols == sel[:, None], -jnp.inf, v)


def _topk(cand, cols):
    return pl.pallas_call(
        _topk_body,
        out_shape=[
            jax.ShapeDtypeStruct((BS, 32), jnp.float32),
            jax.ShapeDtypeStruct((BS, 32), jnp.int32),
        ],
    )(cand, cols)


def _exp_body(nce_ref, e_ref, rs_ref):
    col = jax.lax.broadcasted_iota(jnp.int32, (BS, KP), 1)
    v = jnp.where(col <= K, nce_ref[...], -jnp.inf)
    e = jnp.exp(v * jnp.float32(1.0 / T))
    e_ref[...] = e
    rs_ref[...] = jnp.sum(e, axis=1, keepdims=True)


def _exp_norm(nce_pad):
    return pl.pallas_call(
        _exp_body,
        out_shape=[
            jax.ShapeDtypeStruct((BS, KP), jnp.float32),
            jax.ShapeDtypeStruct((BS, 1), jnp.float32),
        ],
    )(nce_pad)


def _norm_body(my_ref, xw_ref, o_ref):
    w = my_ref[...] * jnp.float32(MOMENTUM) + xw_ref[...] * jnp.float32(1.0 - MOMENTUM)
    n = jnp.maximum(jnp.sqrt(jnp.sum(w * w, axis=1, keepdims=True)), 1e-12)
    o_ref[...] = w / n


def _mix_norm(mem_y, xw):
    return pl.pallas_call(
        _norm_body,
        out_shape=jax.ShapeDtypeStruct((BS, IN), jnp.float32),
    )(mem_y, xw)


NW = 32
ROWS_PER_W = BS // NW
FLAT = NT * BS * TN
NADDR = KP // 128


def _sc_mesh():
    return plsc.VectorSubcoreMesh(core_axis_name="c", subcore_axis_name="s")


def _wid():
    return lax.axis_index("s") * 2 + lax.axis_index("c")


def _iota16():
    return lax.iota(jnp.int32, 16)


def _cand_gather(table, rows3d):
    @functools.partial(
        pl.kernel,
        out_type=jax.ShapeDtypeStruct((BS * 32, 128), jnp.float32),
        mesh=_sc_mesh(),
        scratch_types=[
            pltpu.VMEM((8, 128), jnp.int32),
            pltpu.VMEM((128, 128), jnp.float32),
            pltpu.SemaphoreType.DMA,
        ],
    )
    def k(tab, ridx, out, idx_v, buf_v, sem):
        w = _wid()
        pltpu.sync_copy(ridx.at[w], idx_v)

        def body(s, carry):
            pltpu.async_copy(tab.at[idx_v.at[s]], buf_v, sem).wait()
            pltpu.sync_copy(buf_v, out.at[pl.ds(w * 1024 + s * 128, 128)])
            return carry

        lax.fori_loop(0, 8, body, 0)

    return k(table, rows3d)


def _nce_gather(table_flat, idx_flat):
    @functools.partial(
        pl.kernel,
        out_type=jax.ShapeDtypeStruct((BS, KP), jnp.float32),
        mesh=_sc_mesh(),
        scratch_types=[
            pltpu.VMEM((KP,), jnp.int32),
            pltpu.VMEM((NADDR, 128), jnp.int32),
            pltpu.VMEM((KP,), jnp.float32),
            pltpu.SemaphoreType.DMA,
        ],
    )
    def k(tab, idx_hbm, out, col_v, addr_v, val_v, sem):
        w = _wid()

        def row_body(r, carry):
            b = w * ROWS_PER_W + r
            pltpu.sync_copy(idx_hbm.at[pl.ds(b * KP, KP)], col_v)

            def addr_chunk(j, c2):
                for o in range(8):
                    col = col_v[pl.ds(j * 128 + o * 16, 16)]
                    t = lax.shift_right_arithmetic(col, 11)
                    cc = lax.bitwise_and(col, TN - 1)
                    f = lax.shift_left(t, 21) + (b * TN + cc)
                    addr_v[j, pl.ds(o * 16, 16)] = f
                return c2

            lax.fori_loop(0, NADDR, addr_chunk, 0)

            def fire(j, c2):
                pltpu.async_copy(
                    tab.at[addr_v.at[j]], val_v.at[pl.ds(j * 128, 128)], sem)
                return c2

            lax.fori_loop(0, NADDR, fire, 0)

            def drain(j, c2):
                pltpu.make_async_copy(
                    tab.at[addr_v.at[j]], val_v.at[pl.ds(j * 128, 128)], sem
                ).wait()
                return c2

            lax.fori_loop(0, NADDR, drain, 0)
            pltpu.sync_copy(val_v, out.at[b])
            return carry

        lax.fori_loop(0, ROWS_PER_W, row_body, 0)

    return k(table_flat, idx_flat)


def _label_gather(trainLabel, yi3d):
    @functools.partial(
        pl.kernel,
        out_type=jax.ShapeDtypeStruct((BS * 32,), jnp.int32),
        mesh=_sc_mesh(),
        scratch_types=[
            pltpu.VMEM((8, 128), jnp.int32),
            pltpu.VMEM((128,), jnp.int32),
            pltpu.SemaphoreType.DMA,
        ],
    )
    def k(tab, ridx, out, idx_v, buf_v, sem):
        w = _wid()
        pltpu.sync_copy(ridx.at[w], idx_v)

        def body(s, carry):
            pltpu.async_copy(tab.at[idx_v.at[s]], buf_v, sem).wait()
            pltpu.sync_copy(buf_v, out.at[pl.ds(w * 1024 + s * 128, 128)])
            return carry

        lax.fori_loop(0, 8, body, 0)

    return k(trainLabel, yi3d)


def _update_gathers(memory, x, ysort, wsort):
    @functools.partial(
        pl.kernel,
        out_type=[
            jax.ShapeDtypeStruct((BS, IN), jnp.float32),
            jax.ShapeDtypeStruct((BS, IN), jnp.float32),
        ],
        mesh=_sc_mesh(),
        scratch_types=[
            pltpu.VMEM((ROWS_PER_W,), jnp.int32),
            pltpu.VMEM((ROWS_PER_W, IN), jnp.float32),
            pltpu.SemaphoreType.DMA,
        ],
    )
    def k(mem, xx, ys, ws, out_my, out_xw, idx_v, buf_v, sem):
        w = _wid()
        base = w * ROWS_PER_W
        pltpu.sync_copy(ys.at[pl.ds(base, ROWS_PER_W)], idx_v)
        pltpu.async_copy(mem.at[idx_v], buf_v, sem).wait()
        pltpu.sync_copy(buf_v, out_my.at[pl.ds(base, ROWS_PER_W)])
        pltpu.sync_copy(ws.at[pl.ds(base, ROWS_PER_W)], idx_v)
        pltpu.async_copy(xx.at[idx_v], buf_v, sem).wait()
        pltpu.sync_copy(buf_v, out_xw.at[pl.ds(base, ROWS_PER_W)])

    return k(memory, x, ysort, wsort)


def _update_scatter(new_mem_ref, normed, y):
    @functools.partial(
        pl.kernel,
        out_type=(),
        mesh=_sc_mesh(),
        scratch_types=[
            pltpu.VMEM((ROWS_PER_W,), jnp.int32),
            pltpu.VMEM((ROWS_PER_W, IN), jnp.float32),
            pltpu.SemaphoreType.DMA,
        ],
    )
    def k(nrm, yy, out, idx_v, buf_v, sem):
        w = _wid()
        base = w * ROWS_PER_W
        pltpu.sync_copy(yy.at[pl.ds(base, ROWS_PER_W)], idx_v)
        pltpu.sync_copy(nrm.at[pl.ds(base, ROWS_PER_W)], buf_v)
        pltpu.async_copy(buf_v, out.at[idx_v], sem).wait()

    k(normed, y, new_mem_ref)


def kernel(x, target, y, idx, trainLabel, memory):
    iarange = jnp.arange(BS, dtype=jnp.int32)
    winner = jnp.argmax(jnp.where(y[None, :] == y[:, None], iarange[None, :], -1),
                        axis=1).astype(jnp.int32)

    mem_y, xw = _update_gathers(memory, x, y, winner)
    normed = _mix_norm(mem_y, xw)
    new_mem_ref = jax.new_ref(memory)
    _update_scatter(new_mem_ref, normed, y)
    new_memory = new_mem_ref[...]

    out_full, cmax = _similarity(x, memory)

    chunk_ids = _topchunks(cmax)
    cid_t = chunk_ids.T

    tt = cid_t // NCHT
    ci = cid_t % NCHT
    rows = (tt * BS + iarange[:, None]) * NCHT + ci
    cand = _cand_gather(out_full, rows.reshape(NW, 8, 128))
    cand = cand.reshape(BS, 32 * 128)
    cols = (cid_t[:, :, None] * 128
            + jnp.arange(128, dtype=jnp.int32)[None, None, :]).reshape(BS, 32 * 128)

    yd, yi = _topk(cand, cols)
    retrieval = _label_gather(trainLabel, yi.reshape(NW, 8, 128)).reshape(BS, 32)

    idx_flat = jnp.pad(idx, ((0, 0), (0, KP - (K + 1)))).reshape(BS * KP)
    nce_pad = _nce_gather(out_full.reshape(FLAT), idx_flat)

    e, rowsum = _exp_norm(nce_pad)
    total = jnp.sum(rowsum)
    Z = total / jnp.float32(BS * (K + 1)) * jnp.float32(OUT)
    out = e[:, :K + 1] / Z
    probs = jnp.mean(e[:, 0] / rowsum[:, 0])

    return out, probs, yd, retrieval, new_memory
